# baseline (speedup 1.0000x reference)
import functools

import jax
import jax.numpy as jnp
from jax.experimental import pallas as pl
from jax.experimental.pallas import tpu as pltpu

_TOPK = 60
_THRESHOLD = 0.15
_NMS_THRESHOLD = 0.05
_NMS_TOPK = 20
_STRIDE = 4.0
_D = 24
_N = _D * _D * _D
_ROWS = 108
_ROWS_PAD = 112
_LANES = 128
_BS = 16
_NEG_BIG = -1e30
_IDX_BIG = 1 << 30


_POOL = 6


def _row_top(s, vth, lth, lane3, riota2):
    filt = (s < vth[:, :, None]) | ((s == vth[:, :, None])
                                    & (lane3 > lth[:, :, None]))
    cand = jnp.where(filt, s, -1.0)
    rv = jnp.max(cand, axis=2)
    lw = jnp.where(cand == rv[:, :, None], lane3, _IDX_BIG)
    lmin = jnp.min(lw, axis=2)
    return rv, lmin, riota2 * _LANES + lmin


def _body(logit_ref, shp_ref, off_ref, out_ref, scores_ref,
          pool_v, pool_i, vth_ref, lth_ref, tk_ref):
    x = logit_ref[...]
    scores_ref[...] = 1.0 / (1.0 + jnp.exp(-x))
    lane3 = jax.lax.broadcasted_iota(jnp.int32, (_BS, _ROWS_PAD, _LANES), 2)
    riota2 = jax.lax.broadcasted_iota(jnp.int32, (_BS, _ROWS_PAD), 1)

    s = scores_ref[...]
    vt = jnp.full((_BS, _ROWS_PAD), 2.0, jnp.float32)
    lt = jnp.full((_BS, _ROWS_PAD), -1, jnp.int32)
    for k in range(_POOL):
        rv, lmin, flat = _row_top(s, vt, lt, lane3, riota2)
        pool_v[k] = rv
        pool_i[k] = flat
        vt, lt = rv, lmin
    vth_ref[...] = vt
    lth_ref[...] = lt
    tk_ref[...] = jnp.zeros((_BS, _ROWS_PAD), jnp.int32)

    lane64 = jax.lax.broadcasted_iota(jnp.int32, (_BS, 64), 1)

    def extract(it, carry):
        acc_s, acc_n = carry
        p = pool_v[...]
        pi = pool_i[...]
        m = jnp.max(jnp.max(p, axis=0), axis=1, keepdims=True)
        iw = jnp.where(p == m[None, :, :], pi, _IDX_BIG)
        im = jnp.min(jnp.min(iw, axis=0), axis=1, keepdims=True)
        pool_v[...] = jnp.where(pi == im[None, :, :], -1.0, p)

        oh = lane64 == it
        acc_s = acc_s + jnp.where(oh, m, 0.0)
        acc_n = acc_n + jnp.where(oh, im, 0)

        rsel = im // _LANES
        rowsel = riota2 == rsel
        tk = tk_ref[...] + rowsel.astype(jnp.int32)
        tk_ref[...] = tk
        need = rowsel & (tk == _POOL)
        need_any = jnp.sum(need.astype(jnp.int32)) > 0

        @pl.when(need_any)
        def _refill():
            lpop = im - rsel * _LANES
            vt2 = jnp.where(need, m, vth_ref[...])
            lt2 = jnp.where(need, lpop, lth_ref[...])
            tk_ref[...] = jnp.where(need, 0, tk)
            s2 = scores_ref[...]
            for k in range(_POOL):
                rv, lmin, flat = _row_top(s2, vt2, lt2, lane3, riota2)
                pool_v[k] = jnp.where(need, rv, pool_v[k])
                pool_i[k] = jnp.where(need, flat, pool_i[k])
                vt2 = jnp.where(need, rv, vt2)
                lt2 = jnp.where(need, lmin, lt2)
            vth_ref[...] = vt2
            lth_ref[...] = lt2

        return acc_s, acc_n

    acc_s, acc_n = jax.lax.fori_loop(
        0, _TOPK, extract,
        (jnp.zeros((_BS, 64), jnp.float32), jnp.zeros((_BS, 64), jnp.int32)))

    r = acc_n // _LANES
    l = acc_n - r * _LANES
    z = acc_n // (_D * _D)
    rem = acc_n - z * (_D * _D)
    y = rem // _D
    xx = rem - y * _D

    ohr = (jax.lax.broadcasted_iota(jnp.int32, (_BS, 64, _ROWS_PAD), 2)
           == r[:, :, None]).astype(jnp.float32)
    big = jnp.concatenate(
        [off_ref[:, 0], off_ref[:, 1], off_ref[:, 2],
         shp_ref[:, 0], shp_ref[:, 1], shp_ref[:, 2]], axis=2)
    rowdata = jax.lax.dot_general(
        ohr, big, (((2,), (1,)), ((0,), (0,))),
        precision=jax.lax.Precision.HIGHEST,
        preferred_element_type=jnp.float32)
    ohl = (jax.lax.broadcasted_iota(jnp.int32, (_BS, 64, _LANES), 2)
           == l[:, :, None]).astype(jnp.float32)

    def pick(c):
        return jnp.sum(rowdata[:, :, c * _LANES:(c + 1) * _LANES] * ohl,
                       axis=2)

    ovz, ovy, ovx = pick(0), pick(1), pick(2)
    shz, shy, shx = pick(3), pick(4), pick(5)
    acc_cz = (z.astype(jnp.float32) + ovz) * _STRIDE
    acc_cy = (y.astype(jnp.float32) + ovy) * _STRIDE
    acc_cx = (xx.astype(jnp.float32) + ovx) * _STRIDE
    acc_dz = 2.0 * shz
    acc_dy = 2.0 * shy
    acc_dx = 2.0 * shx

    s_all = acc_s[:, 0:_TOPK]
    cz = acc_cz[:, 0:_TOPK]
    cy = acc_cy[:, 0:_TOPK]
    cx = acc_cx[:, 0:_TOPK]
    dz = acc_dz[:, 0:_TOPK]
    dy = acc_dy[:, 0:_TOPK]
    dx = acc_dx[:, 0:_TOPK]

    loz, hiz = cz - dz / 2.0, cz + dz / 2.0
    loy, hiy = cy - dy / 2.0, cy + dy / 2.0
    lox, hix = cx - dx / 2.0, cx + dx / 2.0
    vol = (jnp.maximum(dz, 0.0) * jnp.maximum(dy, 0.0)) * jnp.maximum(dx, 0.0)

    lane = jax.lax.broadcasted_iota(jnp.int32, (_BS, _TOPK), 1)
    sup = jnp.zeros((_BS, _TOPK), dtype=jnp.bool_)
    keep = jnp.zeros((_BS, _TOPK), dtype=jnp.bool_)
    cnt = jnp.zeros((_BS, 1), dtype=jnp.int32)

    for i in range(_TOPK):
        ci = slice(i, i + 1)
        valid_i = s_all[:, ci] > _THRESHOLD
        take = valid_i & jnp.logical_not(sup[:, ci]) & (cnt < _NMS_TOPK)
        cnt = cnt + take.astype(jnp.int32)
        do_sup = take & (cnt < _NMS_TOPK)

        iz = jnp.maximum(jnp.minimum(hiz, hiz[:, ci]) -
                         jnp.maximum(loz, loz[:, ci]), 0.0)
        iy = jnp.maximum(jnp.minimum(hiy, hiy[:, ci]) -
                         jnp.maximum(loy, loy[:, ci]), 0.0)
        ix = jnp.maximum(jnp.minimum(hix, hix[:, ci]) -
                         jnp.maximum(lox, lox[:, ci]), 0.0)
        inter = (iz * iy) * ix
        union = (vol[:, ci] + vol) - inter
        iou = jnp.where(union > 0.0,
                        inter / jnp.maximum(union, 1e-12), 0.0)

        is_i = lane == i
        keep = keep | (take & is_i)
        sup = sup | (do_sup & ((iou > _NMS_THRESHOLD) | is_i))

    keepI = keep.astype(jnp.int32)
    r_io = jax.lax.broadcasted_iota(jnp.int32, (_BS, _TOPK, _TOPK), 1)
    i_io = jax.lax.broadcasted_iota(jnp.int32, (_BS, _TOPK, _TOPK), 2)
    tri = (i_io <= r_io).astype(jnp.int32)
    kr = jnp.sum(tri * keepI[:, None, :], axis=2)
    rank = kr - 1
    oh = (keep[:, None, :] & (rank[:, None, :] == r_io)).astype(jnp.float32)

    def compact(v):
        return jnp.sum(oh * v[:, None, :], axis=2)

    row_valid = lane < cnt

    def fill(v):
        return jnp.where(row_valid, v, -1.0)

    out_ref[0] = jnp.where(row_valid, 1.0, -1.0)
    out_ref[1] = fill(compact(s_all))
    out_ref[2] = fill(compact(cz))
    out_ref[3] = fill(compact(cy))
    out_ref[4] = fill(compact(cx))
    out_ref[5] = fill(compact(dz))
    out_ref[6] = fill(compact(dy))
    out_ref[7] = fill(compact(dx))


@functools.partial(jax.jit, static_argnums=())
def kernel(Cls, Shape, Offset):
    bs = Cls.shape[0]
    logits = Cls.reshape(bs, _ROWS, _LANES)
    logits = jnp.pad(logits, ((0, 0), (0, _ROWS_PAD - _ROWS), (0, 0)),
                     constant_values=_NEG_BIG)
    shp = Shape.reshape(bs, 3, _ROWS, _LANES)
    shp = jnp.pad(shp, ((0, 0), (0, 0), (0, _ROWS_PAD - _ROWS), (0, 0)))
    off = Offset.reshape(bs, 3, _ROWS, _LANES)
    off = jnp.pad(off, ((0, 0), (0, 0), (0, _ROWS_PAD - _ROWS), (0, 0)))

    out = pl.pallas_call(
        _body,
        out_shape=jax.ShapeDtypeStruct((8, _BS, _TOPK), jnp.float32),
        scratch_shapes=[
            pltpu.VMEM((_BS, _ROWS_PAD, _LANES), jnp.float32),
            pltpu.VMEM((_POOL, _BS, _ROWS_PAD), jnp.float32),
            pltpu.VMEM((_POOL, _BS, _ROWS_PAD), jnp.int32),
            pltpu.VMEM((_BS, _ROWS_PAD), jnp.float32),
            pltpu.VMEM((_BS, _ROWS_PAD), jnp.int32),
            pltpu.VMEM((_BS, _ROWS_PAD), jnp.int32),
        ],
    )(logits, shp, off)
    return jnp.transpose(out, (1, 2, 0))


# speedup vs baseline: 1.4020x; 1.4020x over previous
import functools

import jax
import jax.numpy as jnp
from jax.experimental import pallas as pl
from jax.experimental.pallas import tpu as pltpu

_TOPK = 60
_THRESHOLD = 0.15
_NMS_THRESHOLD = 0.05
_NMS_TOPK = 20
_STRIDE = 4.0
_D = 24
_N = _D * _D * _D
_ROWS = 108
_ROWS_PAD = 112
_LANES = 128
_BS = 16
_NEG_BIG = -1e30
_IDX_BIG = 1 << 30


_POOL = 6


def _row_top(s, vth, lth, lane3, riota2):
    filt = (s < vth[:, :, None]) | ((s == vth[:, :, None])
                                    & (lane3 > lth[:, :, None]))
    cand = jnp.where(filt, s, -1.0)
    rv = jnp.max(cand, axis=2)
    lw = jnp.where(cand == rv[:, :, None], lane3, _IDX_BIG)
    lmin = jnp.min(lw, axis=2)
    return rv, lmin, riota2 * _LANES + lmin


def _body(logit_ref, shp_ref, off_ref, out_ref, scores_ref,
          pool_v, pool_i, vth_ref, lth_ref, tk_ref):
    x = logit_ref[...]
    scores_ref[...] = 1.0 / (1.0 + jnp.exp(-x))
    lane3 = jax.lax.broadcasted_iota(jnp.int32, (_BS, _ROWS_PAD, _LANES), 2)
    riota2 = jax.lax.broadcasted_iota(jnp.int32, (_BS, _ROWS_PAD), 1)

    s = scores_ref[...]
    vt = jnp.full((_BS, _ROWS_PAD), 2.0, jnp.float32)
    lt = jnp.full((_BS, _ROWS_PAD), -1, jnp.int32)
    for k in range(_POOL):
        rv, lmin, flat = _row_top(s, vt, lt, lane3, riota2)
        pool_v[k] = rv
        pool_i[k] = flat
        vt, lt = rv, lmin
    vth_ref[...] = vt
    lth_ref[...] = lt
    tk_ref[...] = jnp.zeros((_BS, _ROWS_PAD), jnp.int32)

    lane64 = jax.lax.broadcasted_iota(jnp.int32, (_BS, 64), 1)

    def extract(it, carry):
        acc_s, acc_n = carry
        p = pool_v[...]
        pi = pool_i[...]
        m = jnp.max(jnp.max(p, axis=0), axis=1, keepdims=True)
        iw = jnp.where(p == m[None, :, :], pi, _IDX_BIG)
        im = jnp.min(jnp.min(iw, axis=0), axis=1, keepdims=True)
        pool_v[...] = jnp.where(pi == im[None, :, :], -1.0, p)

        oh = lane64 == it
        acc_s = acc_s + jnp.where(oh, m, 0.0)
        acc_n = acc_n + jnp.where(oh, im, 0)

        rsel = im // _LANES
        rowsel = riota2 == rsel
        tk = tk_ref[...] + rowsel.astype(jnp.int32)
        tk_ref[...] = tk
        need = rowsel & (tk == _POOL)
        need_any = jnp.sum(need.astype(jnp.int32)) > 0

        @pl.when(need_any)
        def _refill():
            lpop = im - rsel * _LANES
            vt2 = jnp.where(need, m, vth_ref[...])
            lt2 = jnp.where(need, lpop, lth_ref[...])
            tk_ref[...] = jnp.where(need, 0, tk)
            s2 = scores_ref[...]
            for k in range(_POOL):
                rv, lmin, flat = _row_top(s2, vt2, lt2, lane3, riota2)
                pool_v[k] = jnp.where(need, rv, pool_v[k])
                pool_i[k] = jnp.where(need, flat, pool_i[k])
                vt2 = jnp.where(need, rv, vt2)
                lt2 = jnp.where(need, lmin, lt2)
            vth_ref[...] = vt2
            lth_ref[...] = lt2

        return acc_s, acc_n

    acc_s, acc_n = jax.lax.fori_loop(
        0, _TOPK, extract,
        (jnp.zeros((_BS, 64), jnp.float32), jnp.zeros((_BS, 64), jnp.int32)))

    r = acc_n // _LANES
    l = acc_n - r * _LANES
    z = acc_n // (_D * _D)
    rem = acc_n - z * (_D * _D)
    y = rem // _D
    xx = rem - y * _D

    ohr = (jax.lax.broadcasted_iota(jnp.int32, (_BS, 64, _ROWS_PAD), 2)
           == r[:, :, None]).astype(jnp.float32)
    big = jnp.concatenate(
        [off_ref[:, 0], off_ref[:, 1], off_ref[:, 2],
         shp_ref[:, 0], shp_ref[:, 1], shp_ref[:, 2]], axis=2)
    rowdata = jax.lax.dot_general(
        ohr, big, (((2,), (1,)), ((0,), (0,))),
        precision=jax.lax.Precision.HIGHEST,
        preferred_element_type=jnp.float32)
    ohl = (jax.lax.broadcasted_iota(jnp.int32, (_BS, 64, _LANES), 2)
           == l[:, :, None]).astype(jnp.float32)

    def pick(c):
        return jnp.sum(rowdata[:, :, c * _LANES:(c + 1) * _LANES] * ohl,
                       axis=2)

    ovz, ovy, ovx = pick(0), pick(1), pick(2)
    shz, shy, shx = pick(3), pick(4), pick(5)
    acc_cz = (z.astype(jnp.float32) + ovz) * _STRIDE
    acc_cy = (y.astype(jnp.float32) + ovy) * _STRIDE
    acc_cx = (xx.astype(jnp.float32) + ovx) * _STRIDE
    acc_dz = 2.0 * shz
    acc_dy = 2.0 * shy
    acc_dx = 2.0 * shx

    s_all = acc_s[:, 0:_TOPK]
    cz = acc_cz[:, 0:_TOPK]
    cy = acc_cy[:, 0:_TOPK]
    cx = acc_cx[:, 0:_TOPK]
    dz = acc_dz[:, 0:_TOPK]
    dy = acc_dy[:, 0:_TOPK]
    dx = acc_dx[:, 0:_TOPK]

    loz, hiz = cz - dz / 2.0, cz + dz / 2.0
    loy, hiy = cy - dy / 2.0, cy + dy / 2.0
    lox, hix = cx - dx / 2.0, cx + dx / 2.0
    vol = (jnp.maximum(dz, 0.0) * jnp.maximum(dy, 0.0)) * jnp.maximum(dx, 0.0)

    def pair(lo, hi):
        return jnp.maximum(jnp.minimum(hi[:, :, None], hi[:, None, :]) -
                           jnp.maximum(lo[:, :, None], lo[:, None, :]), 0.0)

    inter = (pair(loz, hiz) * pair(loy, hiy)) * pair(lox, hix)
    union = (vol[:, :, None] + vol[:, None, :]) - inter
    iou = jnp.where(union > 0.0, inter / jnp.maximum(union, 1e-12), 0.0)
    d_io = jax.lax.broadcasted_iota(jnp.int32, (_BS, _TOPK, _TOPK), 1)
    d_jo = jax.lax.broadcasted_iota(jnp.int32, (_BS, _TOPK, _TOPK), 2)
    adj = (iou > _NMS_THRESHOLD) | (d_io == d_jo)

    lane = jax.lax.broadcasted_iota(jnp.int32, (_BS, _TOPK), 1)
    valid = s_all > _THRESHOLD
    sup = jnp.zeros((_BS, _TOPK), dtype=jnp.bool_)
    keep = jnp.zeros((_BS, _TOPK), dtype=jnp.bool_)
    cnt = jnp.zeros((_BS, 1), dtype=jnp.int32)

    for i in range(_TOPK):
        ci = slice(i, i + 1)
        take = valid[:, ci] & jnp.logical_not(sup[:, ci]) & (cnt < _NMS_TOPK)
        cnt = cnt + take.astype(jnp.int32)
        do_sup = take & (cnt < _NMS_TOPK)
        keep = keep | (take & (lane == i))
        sup = sup | (do_sup & adj[:, i, :])

    keepI = keep.astype(jnp.int32)
    r_io = jax.lax.broadcasted_iota(jnp.int32, (_BS, _TOPK, _TOPK), 1)
    i_io = jax.lax.broadcasted_iota(jnp.int32, (_BS, _TOPK, _TOPK), 2)
    tri = (i_io <= r_io).astype(jnp.int32)
    kr = jnp.sum(tri * keepI[:, None, :], axis=2)
    rank = kr - 1
    oh = (keep[:, None, :] & (rank[:, None, :] == r_io)).astype(jnp.float32)

    def compact(v):
        return jnp.sum(oh * v[:, None, :], axis=2)

    row_valid = lane < cnt

    def fill(v):
        return jnp.where(row_valid, v, -1.0)

    out_ref[0] = jnp.where(row_valid, 1.0, -1.0)
    out_ref[1] = fill(compact(s_all))
    out_ref[2] = fill(compact(cz))
    out_ref[3] = fill(compact(cy))
    out_ref[4] = fill(compact(cx))
    out_ref[5] = fill(compact(dz))
    out_ref[6] = fill(compact(dy))
    out_ref[7] = fill(compact(dx))


@functools.partial(jax.jit, static_argnums=())
def kernel(Cls, Shape, Offset):
    bs = Cls.shape[0]
    logits = Cls.reshape(bs, _ROWS, _LANES)
    logits = jnp.pad(logits, ((0, 0), (0, _ROWS_PAD - _ROWS), (0, 0)),
                     constant_values=_NEG_BIG)
    shp = Shape.reshape(bs, 3, _ROWS, _LANES)
    shp = jnp.pad(shp, ((0, 0), (0, 0), (0, _ROWS_PAD - _ROWS), (0, 0)))
    off = Offset.reshape(bs, 3, _ROWS, _LANES)
    off = jnp.pad(off, ((0, 0), (0, 0), (0, _ROWS_PAD - _ROWS), (0, 0)))

    out = pl.pallas_call(
        _body,
        out_shape=jax.ShapeDtypeStruct((8, _BS, _TOPK), jnp.float32),
        scratch_shapes=[
            pltpu.VMEM((_BS, _ROWS_PAD, _LANES), jnp.float32),
            pltpu.VMEM((_POOL, _BS, _ROWS_PAD), jnp.float32),
            pltpu.VMEM((_POOL, _BS, _ROWS_PAD), jnp.int32),
            pltpu.VMEM((_BS, _ROWS_PAD), jnp.float32),
            pltpu.VMEM((_BS, _ROWS_PAD), jnp.int32),
            pltpu.VMEM((_BS, _ROWS_PAD), jnp.int32),
        ],
    )(logits, shp, off)
    return jnp.transpose(out, (1, 2, 0))


# speedup vs baseline: 1.5443x; 1.1015x over previous
import functools

import jax
import jax.numpy as jnp
from jax.experimental import pallas as pl
from jax.experimental.pallas import tpu as pltpu

_TOPK = 60
_THRESHOLD = 0.15
_NMS_THRESHOLD = 0.05
_NMS_TOPK = 20
_STRIDE = 4.0
_D = 24
_N = _D * _D * _D
_ROWS = 108
_ROWS_PAD = 112
_LANES = 128
_BS = 16
_NEG_BIG = -1e30
_IDX_BIG = 1 << 30


_POOL = 6


def _row_top(s, vth, lth, lane3, riota3):
    filt = (s < vth) | ((s == vth) & (lane3 > lth))
    cand = jnp.where(filt, s, -1.0)
    rv = jnp.max(cand, axis=2, keepdims=True)
    lw = jnp.where(cand == rv, lane3, _IDX_BIG)
    lmin = jnp.min(lw, axis=2, keepdims=True)
    return rv, lmin, riota3 * _LANES + lmin


def _body(logit_ref, shp_ref, off_ref, out_ref, scores_ref,
          pool_v, pool_i, vth_ref, lth_ref, tk_ref):
    x = logit_ref[...]
    scores_ref[...] = 1.0 / (1.0 + jnp.exp(-x))
    lane3 = jax.lax.broadcasted_iota(jnp.int32, (_BS, _ROWS_PAD, _LANES), 2)
    riota2 = jax.lax.broadcasted_iota(jnp.int32, (_BS, _ROWS_PAD), 1)
    riota3 = jax.lax.broadcasted_iota(jnp.int32, (_BS, _ROWS_PAD, 1), 1)

    s = scores_ref[...]
    vt = jnp.full((_BS, _ROWS_PAD, 1), 2.0, jnp.float32)
    lt = jnp.full((_BS, _ROWS_PAD, 1), -1, jnp.int32)
    for k in range(_POOL):
        rv, lmin, flat = _row_top(s, vt, lt, lane3, riota3)
        pool_v[k] = rv.reshape(_BS, _ROWS_PAD)
        pool_i[k] = flat.reshape(_BS, _ROWS_PAD)
        vt, lt = rv, lmin
    vth_ref[...] = vt
    lth_ref[...] = lt
    tk_ref[...] = jnp.zeros((_BS, _ROWS_PAD), jnp.int32)

    lane64 = jax.lax.broadcasted_iota(jnp.int32, (_BS, 64), 1)

    def extract(it, carry):
        acc_s, acc_n = carry
        p = pool_v[...]
        pi = pool_i[...]
        m = jnp.max(jnp.max(p, axis=0), axis=1, keepdims=True)
        iw = jnp.where(p == m[None, :, :], pi, _IDX_BIG)
        im = jnp.min(jnp.min(iw, axis=0), axis=1, keepdims=True)
        pool_v[...] = jnp.where(pi == im[None, :, :], -1.0, p)

        oh = lane64 == it
        acc_s = acc_s + jnp.where(oh, m, 0.0)
        acc_n = acc_n + jnp.where(oh, im, 0)

        rsel = im // _LANES
        rowsel = riota2 == rsel
        tk = tk_ref[...] + rowsel.astype(jnp.int32)
        tk_ref[...] = tk
        need = rowsel & (tk == _POOL)
        need_any = jnp.sum(need.astype(jnp.int32)) > 0

        @pl.when(need_any)
        def _refill():
            lpop = im - rsel * _LANES
            need3 = need.astype(jnp.int32)[:, :, None] == 1
            vt2 = jnp.where(need3, m[:, :, None], vth_ref[...])
            lt2 = jnp.where(need3, lpop[:, :, None], lth_ref[...])
            tk_ref[...] = jnp.where(need, 0, tk)
            s2 = scores_ref[...]
            for k in range(_POOL):
                rv, lmin, flat = _row_top(s2, vt2, lt2, lane3, riota3)
                pool_v[k] = jnp.where(need, rv.reshape(_BS, _ROWS_PAD),
                                      pool_v[k])
                pool_i[k] = jnp.where(need, flat.reshape(_BS, _ROWS_PAD),
                                      pool_i[k])
                vt2 = jnp.where(need3, rv, vt2)
                lt2 = jnp.where(need3, lmin, lt2)
            vth_ref[...] = vt2
            lth_ref[...] = lt2

        return acc_s, acc_n

    acc_s, acc_n = jax.lax.fori_loop(
        0, _TOPK, extract,
        (jnp.zeros((_BS, 64), jnp.float32), jnp.zeros((_BS, 64), jnp.int32)))

    r = acc_n // _LANES
    l = acc_n - r * _LANES
    z = acc_n // (_D * _D)
    rem = acc_n - z * (_D * _D)
    y = rem // _D
    xx = rem - y * _D

    ohr = (jax.lax.broadcasted_iota(jnp.int32, (_BS, 64, _ROWS_PAD), 2)
           == r[:, :, None]).astype(jnp.float32)
    big = jnp.concatenate(
        [off_ref[:, 0], off_ref[:, 1], off_ref[:, 2],
         shp_ref[:, 0], shp_ref[:, 1], shp_ref[:, 2]], axis=2)
    rowdata = jax.lax.dot_general(
        ohr, big, (((2,), (1,)), ((0,), (0,))),
        precision=jax.lax.Precision.HIGHEST,
        preferred_element_type=jnp.float32)
    ohl = (jax.lax.broadcasted_iota(jnp.int32, (_BS, 64, _LANES), 2)
           == l[:, :, None]).astype(jnp.float32)

    def pick(c):
        return jnp.sum(rowdata[:, :, c * _LANES:(c + 1) * _LANES] * ohl,
                       axis=2)

    ovz, ovy, ovx = pick(0), pick(1), pick(2)
    shz, shy, shx = pick(3), pick(4), pick(5)
    acc_cz = (z.astype(jnp.float32) + ovz) * _STRIDE
    acc_cy = (y.astype(jnp.float32) + ovy) * _STRIDE
    acc_cx = (xx.astype(jnp.float32) + ovx) * _STRIDE
    acc_dz = 2.0 * shz
    acc_dy = 2.0 * shy
    acc_dx = 2.0 * shx

    s_all = acc_s[:, 0:_TOPK]
    cz = acc_cz[:, 0:_TOPK]
    cy = acc_cy[:, 0:_TOPK]
    cx = acc_cx[:, 0:_TOPK]
    dz = acc_dz[:, 0:_TOPK]
    dy = acc_dy[:, 0:_TOPK]
    dx = acc_dx[:, 0:_TOPK]

    loz, hiz = cz - dz / 2.0, cz + dz / 2.0
    loy, hiy = cy - dy / 2.0, cy + dy / 2.0
    lox, hix = cx - dx / 2.0, cx + dx / 2.0
    vol = (jnp.maximum(dz, 0.0) * jnp.maximum(dy, 0.0)) * jnp.maximum(dx, 0.0)

    def pair(lo, hi):
        return jnp.maximum(jnp.minimum(hi[:, :, None], hi[:, None, :]) -
                           jnp.maximum(lo[:, :, None], lo[:, None, :]), 0.0)

    inter = (pair(loz, hiz) * pair(loy, hiy)) * pair(lox, hix)
    union = (vol[:, :, None] + vol[:, None, :]) - inter
    iou = jnp.where(union > 0.0, inter / jnp.maximum(union, 1e-12), 0.0)
    d_io = jax.lax.broadcasted_iota(jnp.int32, (_BS, _TOPK, _TOPK), 1)
    d_jo = jax.lax.broadcasted_iota(jnp.int32, (_BS, _TOPK, _TOPK), 2)
    adj = (iou > _NMS_THRESHOLD) | (d_io == d_jo)

    lane = jax.lax.broadcasted_iota(jnp.int32, (_BS, _TOPK), 1)
    valid = s_all > _THRESHOLD
    sup = jnp.zeros((_BS, _TOPK), dtype=jnp.bool_)
    rnk = jnp.zeros((_BS, _TOPK), dtype=jnp.int32)
    cnt = jnp.zeros((_BS, 1), dtype=jnp.int32)

    for i in range(_TOPK):
        ci = slice(i, i + 1)
        take = valid[:, ci] & jnp.logical_not(sup[:, ci]) & (cnt < _NMS_TOPK)
        rnk = rnk + jnp.where(take & (lane == i), cnt + 1, 0)
        cnt = cnt + take.astype(jnp.int32)
        do_sup = take & (cnt < _NMS_TOPK)
        sup = sup | (do_sup & adj[:, i, :])

    rank = rnk - 1
    r_io2 = jax.lax.broadcasted_iota(jnp.int32, (_BS, _TOPK, _TOPK), 2)
    ohT = (rank[:, :, None] == r_io2).astype(jnp.float32)

    def compact(v):
        return jnp.sum(ohT * v[:, :, None], axis=1)

    row_valid = lane < cnt

    def fill(v):
        return jnp.where(row_valid, v, -1.0)

    out_ref[0] = jnp.where(row_valid, 1.0, -1.0)
    out_ref[1] = fill(compact(s_all))
    out_ref[2] = fill(compact(cz))
    out_ref[3] = fill(compact(cy))
    out_ref[4] = fill(compact(cx))
    out_ref[5] = fill(compact(dz))
    out_ref[6] = fill(compact(dy))
    out_ref[7] = fill(compact(dx))


@functools.partial(jax.jit, static_argnums=())
def kernel(Cls, Shape, Offset):
    bs = Cls.shape[0]
    logits = Cls.reshape(bs, _ROWS, _LANES)
    logits = jnp.pad(logits, ((0, 0), (0, _ROWS_PAD - _ROWS), (0, 0)),
                     constant_values=_NEG_BIG)
    shp = Shape.reshape(bs, 3, _ROWS, _LANES)
    shp = jnp.pad(shp, ((0, 0), (0, 0), (0, _ROWS_PAD - _ROWS), (0, 0)))
    off = Offset.reshape(bs, 3, _ROWS, _LANES)
    off = jnp.pad(off, ((0, 0), (0, 0), (0, _ROWS_PAD - _ROWS), (0, 0)))

    out = pl.pallas_call(
        _body,
        out_shape=jax.ShapeDtypeStruct((8, _BS, _TOPK), jnp.float32),
        scratch_shapes=[
            pltpu.VMEM((_BS, _ROWS_PAD, _LANES), jnp.float32),
            pltpu.VMEM((_POOL, _BS, _ROWS_PAD), jnp.float32),
            pltpu.VMEM((_POOL, _BS, _ROWS_PAD), jnp.int32),
            pltpu.VMEM((_BS, _ROWS_PAD, 1), jnp.float32),
            pltpu.VMEM((_BS, _ROWS_PAD, 1), jnp.int32),
            pltpu.VMEM((_BS, _ROWS_PAD), jnp.int32),
        ],
    )(logits, shp, off)
    return jnp.transpose(out, (1, 2, 0))


# speedup vs baseline: 1.5494x; 1.0033x over previous
import functools

import jax
import jax.numpy as jnp
from jax.experimental import pallas as pl
from jax.experimental.pallas import tpu as pltpu

_TOPK = 60
_THRESHOLD = 0.15
_NMS_THRESHOLD = 0.05
_NMS_TOPK = 20
_STRIDE = 4.0
_D = 24
_N = _D * _D * _D
_ROWS = 108
_ROWS_PAD = 112
_LANES = 128
_BS = 16
_NEG_BIG = -1e30
_IDX_BIG = 1 << 30


_POOL = 6


def _row_top(s, vth, lth, lane3, riota3):
    filt = (s < vth) | ((s == vth) & (lane3 > lth))
    cand = jnp.where(filt, s, -1.0)
    rv = jnp.max(cand, axis=2, keepdims=True)
    lw = jnp.where(cand == rv, lane3, _IDX_BIG)
    lmin = jnp.min(lw, axis=2, keepdims=True)
    return rv, lmin, riota3 * _LANES + lmin


def _body(logit_ref, shp_ref, off_ref, out_ref, scores_ref,
          pool_v, pool_i, vth_ref, lth_ref, tk_ref):
    x = logit_ref[...]
    scores_ref[...] = 1.0 / (1.0 + jnp.exp(-x))
    lane3 = jax.lax.broadcasted_iota(jnp.int32, (_BS, _ROWS_PAD, _LANES), 2)
    riota2 = jax.lax.broadcasted_iota(jnp.int32, (_BS, _ROWS_PAD), 1)
    riota3 = jax.lax.broadcasted_iota(jnp.int32, (_BS, _ROWS_PAD, 1), 1)

    sw = scores_ref[...]
    for k in range(_POOL):
        rv = jnp.max(sw, axis=2, keepdims=True)
        lw = jnp.where(sw == rv, lane3, _IDX_BIG)
        lmin = jnp.min(lw, axis=2, keepdims=True)
        pool_v[k] = rv.reshape(_BS, _ROWS_PAD)
        pool_i[k] = (riota3 * _LANES + lmin).reshape(_BS, _ROWS_PAD)
        if k < _POOL - 1:
            sw = jnp.where(lane3 == lmin, -1.0, sw)
        else:
            vth_ref[...] = rv
            lth_ref[...] = lmin
    tk_ref[...] = jnp.zeros((_BS, _ROWS_PAD), jnp.int32)

    lane64 = jax.lax.broadcasted_iota(jnp.int32, (_BS, 64), 1)

    def extract(it, carry):
        acc_s, acc_n = carry
        p = pool_v[...]
        pi = pool_i[...]
        m = jnp.max(jnp.max(p, axis=0), axis=1, keepdims=True)
        iw = jnp.where(p == m[None, :, :], pi, _IDX_BIG)
        im = jnp.min(jnp.min(iw, axis=0), axis=1, keepdims=True)
        pool_v[...] = jnp.where(pi == im[None, :, :], -1.0, p)

        oh = lane64 == it
        acc_s = acc_s + jnp.where(oh, m, 0.0)
        acc_n = acc_n + jnp.where(oh, im, 0)

        rsel = im // _LANES
        rowsel = riota2 == rsel
        tk = tk_ref[...] + rowsel.astype(jnp.int32)
        tk_ref[...] = tk
        need = rowsel & (tk == _POOL)
        need_any = jnp.sum(need.astype(jnp.int32)) > 0

        @pl.when(need_any)
        def _refill():
            lpop = im - rsel * _LANES
            need3 = need.astype(jnp.int32)[:, :, None] == 1
            vt2 = jnp.where(need3, m[:, :, None], vth_ref[...])
            lt2 = jnp.where(need3, lpop[:, :, None], lth_ref[...])
            tk_ref[...] = jnp.where(need, 0, tk)
            s2 = scores_ref[...]
            for k in range(_POOL):
                rv, lmin, flat = _row_top(s2, vt2, lt2, lane3, riota3)
                pool_v[k] = jnp.where(need, rv.reshape(_BS, _ROWS_PAD),
                                      pool_v[k])
                pool_i[k] = jnp.where(need, flat.reshape(_BS, _ROWS_PAD),
                                      pool_i[k])
                vt2 = jnp.where(need3, rv, vt2)
                lt2 = jnp.where(need3, lmin, lt2)
            vth_ref[...] = vt2
            lth_ref[...] = lt2

        return acc_s, acc_n

    acc_s, acc_n = jax.lax.fori_loop(
        0, _TOPK, extract,
        (jnp.zeros((_BS, 64), jnp.float32), jnp.zeros((_BS, 64), jnp.int32)))

    r = acc_n // _LANES
    l = acc_n - r * _LANES
    z = acc_n // (_D * _D)
    rem = acc_n - z * (_D * _D)
    y = rem // _D
    xx = rem - y * _D

    ohr = (jax.lax.broadcasted_iota(jnp.int32, (_BS, 64, _ROWS_PAD), 2)
           == r[:, :, None]).astype(jnp.float32)
    big = jnp.concatenate(
        [off_ref[:, 0], off_ref[:, 1], off_ref[:, 2],
         shp_ref[:, 0], shp_ref[:, 1], shp_ref[:, 2]], axis=2)
    rowdata = jax.lax.dot_general(
        ohr, big, (((2,), (1,)), ((0,), (0,))),
        precision=jax.lax.Precision.HIGHEST,
        preferred_element_type=jnp.float32)
    ohl = (jax.lax.broadcasted_iota(jnp.int32, (_BS, 64, _LANES), 2)
           == l[:, :, None]).astype(jnp.float32)

    def pick(c):
        return jnp.sum(rowdata[:, :, c * _LANES:(c + 1) * _LANES] * ohl,
                       axis=2)

    ovz, ovy, ovx = pick(0), pick(1), pick(2)
    shz, shy, shx = pick(3), pick(4), pick(5)
    acc_cz = (z.astype(jnp.float32) + ovz) * _STRIDE
    acc_cy = (y.astype(jnp.float32) + ovy) * _STRIDE
    acc_cx = (xx.astype(jnp.float32) + ovx) * _STRIDE
    acc_dz = 2.0 * shz
    acc_dy = 2.0 * shy
    acc_dx = 2.0 * shx

    s_all = acc_s[:, 0:_TOPK]
    cz = acc_cz[:, 0:_TOPK]
    cy = acc_cy[:, 0:_TOPK]
    cx = acc_cx[:, 0:_TOPK]
    dz = acc_dz[:, 0:_TOPK]
    dy = acc_dy[:, 0:_TOPK]
    dx = acc_dx[:, 0:_TOPK]

    loz, hiz = cz - dz / 2.0, cz + dz / 2.0
    loy, hiy = cy - dy / 2.0, cy + dy / 2.0
    lox, hix = cx - dx / 2.0, cx + dx / 2.0
    vol = (jnp.maximum(dz, 0.0) * jnp.maximum(dy, 0.0)) * jnp.maximum(dx, 0.0)

    def pair(lo, hi):
        return jnp.maximum(jnp.minimum(hi[:, :, None], hi[:, None, :]) -
                           jnp.maximum(lo[:, :, None], lo[:, None, :]), 0.0)

    inter = (pair(loz, hiz) * pair(loy, hiy)) * pair(lox, hix)
    union = (vol[:, :, None] + vol[:, None, :]) - inter
    iou = jnp.where(union > 0.0, inter / jnp.maximum(union, 1e-12), 0.0)
    d_io = jax.lax.broadcasted_iota(jnp.int32, (_BS, _TOPK, _TOPK), 1)
    d_jo = jax.lax.broadcasted_iota(jnp.int32, (_BS, _TOPK, _TOPK), 2)
    adj = (iou > _NMS_THRESHOLD) | (d_io == d_jo)

    lane = jax.lax.broadcasted_iota(jnp.int32, (_BS, _TOPK), 1)
    valid = s_all > _THRESHOLD
    sup = jnp.zeros((_BS, _TOPK), dtype=jnp.bool_)
    rnk = jnp.zeros((_BS, _TOPK), dtype=jnp.int32)
    cnt = jnp.zeros((_BS, 1), dtype=jnp.int32)

    for i in range(_TOPK):
        ci = slice(i, i + 1)
        take = valid[:, ci] & jnp.logical_not(sup[:, ci]) & (cnt < _NMS_TOPK)
        rnk = rnk + jnp.where(take & (lane == i), cnt + 1, 0)
        cnt = cnt + take.astype(jnp.int32)
        do_sup = take & (cnt < _NMS_TOPK)
        sup = sup | (do_sup & adj[:, i, :])

    rank = rnk - 1
    r_io2 = jax.lax.broadcasted_iota(jnp.int32, (_BS, _TOPK, _TOPK), 2)
    ohT = (rank[:, :, None] == r_io2).astype(jnp.float32)

    def compact(v):
        return jnp.sum(ohT * v[:, :, None], axis=1)

    row_valid = lane < cnt

    def fill(v):
        return jnp.where(row_valid, v, -1.0)

    out_ref[0] = jnp.where(row_valid, 1.0, -1.0)
    out_ref[1] = fill(compact(s_all))
    out_ref[2] = fill(compact(cz))
    out_ref[3] = fill(compact(cy))
    out_ref[4] = fill(compact(cx))
    out_ref[5] = fill(compact(dz))
    out_ref[6] = fill(compact(dy))
    out_ref[7] = fill(compact(dx))


@functools.partial(jax.jit, static_argnums=())
def kernel(Cls, Shape, Offset):
    bs = Cls.shape[0]
    logits = Cls.reshape(bs, _ROWS, _LANES)
    logits = jnp.pad(logits, ((0, 0), (0, _ROWS_PAD - _ROWS), (0, 0)),
                     constant_values=_NEG_BIG)
    shp = Shape.reshape(bs, 3, _ROWS, _LANES)
    shp = jnp.pad(shp, ((0, 0), (0, 0), (0, _ROWS_PAD - _ROWS), (0, 0)))
    off = Offset.reshape(bs, 3, _ROWS, _LANES)
    off = jnp.pad(off, ((0, 0), (0, 0), (0, _ROWS_PAD - _ROWS), (0, 0)))

    out = pl.pallas_call(
        _body,
        out_shape=jax.ShapeDtypeStruct((8, _BS, _TOPK), jnp.float32),
        scratch_shapes=[
            pltpu.VMEM((_BS, _ROWS_PAD, _LANES), jnp.float32),
            pltpu.VMEM((_POOL, _BS, _ROWS_PAD), jnp.float32),
            pltpu.VMEM((_POOL, _BS, _ROWS_PAD), jnp.int32),
            pltpu.VMEM((_BS, _ROWS_PAD, 1), jnp.float32),
            pltpu.VMEM((_BS, _ROWS_PAD, 1), jnp.int32),
            pltpu.VMEM((_BS, _ROWS_PAD), jnp.int32),
        ],
    )(logits, shp, off)
    return jnp.transpose(out, (1, 2, 0))
